# Initial kernel scaffold; baseline (speedup 1.0000x reference)
import jax
import jax.numpy as jnp
from jax.experimental import pallas as pl


def kernel(r_idx, entities_idx, arity, edge_list, rel_list, query_emb, pos_emb, rel_embs, Ws, bs, ln_scales, ln_biases, mlpW1, mlpb1, mlpW2, mlpb2):
    raise NotImplementedError("write your pallas kernel here")


# speedup vs baseline: 7.3009x; 7.3009x over previous
import functools
import jax
import jax.numpy as jnp
from jax import lax
from jax.experimental import pallas as pl
from jax.experimental.pallas import tpu as pltpu
from jax.experimental.pallas import tpu_sc as plsc

D = 128
A = 6
NLANE = 16
NSUB = 16
NCORE = 2
NP = 10112
EP = 20480
CHUNK = 16
RCHUNK = 64
NRCHUNK = NP // RCHUNK


def _sc_layer_body(x_hbm, edgeT_hbm, rel_list_hbm, rel_tab_hbm, pos_hbm,
                   possum_hbm, out_hbm, eidx, idxg, relidx, midx, xrows,
                   relrows, posrows, msgf, pos_v, zbuf, agg_sh, sem):
    c = lax.axis_index("c")
    s = lax.axis_index("s")

    pltpu.sync_copy(pos_hbm, pos_v)

    zvec = jnp.zeros((NLANE,), jnp.float32)

    def zrow(r, carry):
        for d in range(D // NLANE):
            zbuf[r, pl.ds(d * NLANE, NLANE)] = zvec
        return carry

    lax.fori_loop(0, RCHUNK, zrow, 0)
    for t in range((NRCHUNK + NSUB - 1) // NSUB):
        cid = s + t * NSUB

        @pl.when(cid < NRCHUNK)
        def _():
            pltpu.sync_copy(zbuf,
                            agg_sh.at[pl.ds(cid * RCHUNK, RCHUNK)])
    plsc.subcore_barrier()

    e0 = s * (EP // NSUB)
    xoff = c * NP

    def chunk_body(k, carry):
        off = e0 + k * CHUNK
        for j in range(A):
            pltpu.sync_copy(edgeT_hbm.at[j, pl.ds(off, CHUNK)], eidx.at[j])
        pltpu.sync_copy(rel_list_hbm.at[pl.ds(off, CHUNK)], relidx)
        for v in range(CHUNK // NLANE):
            sl = pl.ds(v * NLANE, NLANE)
            mc = jnp.zeros((NLANE,), jnp.int32)
            for j in range(A):
                ej = eidx[j, sl]
                idxg[j, sl] = ej + jnp.full((NLANE,), xoff, jnp.int32)
                mc = mc + jnp.where(ej != 0, 1 << j, 0)
            midx[sl] = mc
        handles = [
            pltpu.async_copy(x_hbm.at[idxg.at[j]], xrows.at[j], sem)
            for j in range(A)
        ]
        handles.append(pltpu.async_copy(rel_tab_hbm.at[relidx], relrows, sem))
        handles.append(pltpu.async_copy(possum_hbm.at[midx], posrows, sem))
        for h in handles:
            h.wait()

        def edge_body(e, carry2):
            srow = []
            rrow = []
            for d in range(D // NLANE):
                sl = pl.ds(d * NLANE, NLANE)
                acc = posrows[e, sl]
                for j in range(A):
                    acc = acc + xrows[j, e, sl]
                srow.append(acc)
                rrow.append(relrows[e, sl])
            for i in range(A):
                for d in range(D // NLANE):
                    sl = pl.ds(d * NLANE, NLANE)
                    tmp = srow[d] - xrows[i, e, sl] - pos_v[i, sl]
                    msgf[i * CHUNK + e, sl] = tmp * rrow[d]
            return carry2

        lax.fori_loop(0, CHUNK, edge_body, 0)

        for i in range(A):
            pltpu.sync_copy(msgf.at[pl.ds(i * CHUNK, CHUNK)],
                            agg_sh.at[eidx.at[i]], add=True)
        return carry

    lax.fori_loop(0, EP // NSUB // CHUNK, chunk_body, 0)
    plsc.subcore_barrier()

    for t in range((NRCHUNK + NSUB - 1) // NSUB):
        cid = s + t * NSUB

        @pl.when(cid < NRCHUNK)
        def _():
            rr = pl.ds(cid * RCHUNK, RCHUNK)
            pltpu.sync_copy(agg_sh.at[rr], zbuf)
            pltpu.sync_copy(zbuf, out_hbm.at[c].at[rr])


def _make_sc_layer():
    mesh = plsc.VectorSubcoreMesh(core_axis_name="c", subcore_axis_name="s")
    return pl.kernel(
        _sc_layer_body,
        mesh=mesh,
        out_type=jax.ShapeDtypeStruct((NCORE, NP, D), jnp.float32),
        scratch_types=[
            pltpu.VMEM((A, CHUNK), jnp.int32),
            pltpu.VMEM((A, CHUNK), jnp.int32),
            pltpu.VMEM((CHUNK,), jnp.int32),
            pltpu.VMEM((CHUNK,), jnp.int32),
            pltpu.VMEM((A, CHUNK, D), jnp.float32),
            pltpu.VMEM((CHUNK, D), jnp.float32),
            pltpu.VMEM((CHUNK, D), jnp.float32),
            pltpu.VMEM((A * CHUNK, D), jnp.float32),
            pltpu.VMEM((A, D), jnp.float32),
            pltpu.VMEM((RCHUNK, D), jnp.float32),
            pltpu.VMEM_SHARED((NP, D), jnp.float32),
            pltpu.SemaphoreType.DMA,
        ],
    )


def _init_body(tgt_ref, vals_ref, o_ref):
    o_ref[...] = jnp.zeros(o_ref.shape, o_ref.dtype)
    B = tgt_ref.shape[0]
    for b in range(B):
        for j in range(A):
            t = tgt_ref[b, j]
            cur = o_ref[b, pl.ds(t, 1), :]
            o_ref[b, pl.ds(t, 1), :] = cur + vals_ref[b, j, :][None, :]


def _dense_body(agg_ref, xp_ref, w_ref, b_ref, sc_ref, bi_ref, o_ref):
    h = jnp.dot(agg_ref[...], w_ref[...],
                preferred_element_type=jnp.float32) + b_ref[...]
    mu = jnp.mean(h, axis=-1, keepdims=True)
    hc = h - mu
    var = jnp.mean(hc * hc, axis=-1, keepdims=True)
    h = hc * lax.rsqrt(var + 1e-5) * sc_ref[...] + bi_ref[...]
    blk = agg_ref.shape[0]
    rowid = (lax.broadcasted_iota(jnp.int32, (blk, 1), 0)
             + pl.program_id(0) * blk)
    keep = ((rowid % NP) != 0).astype(jnp.float32)
    o_ref[...] = (jnp.maximum(h, 0.0) + xp_ref[...]) * keep


def _head_body(coll_ref, x_ref, q_ref, w1_ref, b1_ref, w2_ref, o_ref, xg_ref):
    B, NC = coll_ref.shape
    for b in range(B):
        for n in range(NC):
            t = coll_ref[b, n]
            xg_ref[pl.ds(b * NC + n, 1), :] = x_ref[b, pl.ds(t, 1), :]
    h1 = jnp.dot(xg_ref[...], w1_ref[:D, :], preferred_element_type=jnp.float32)
    qc = jnp.dot(q_ref[...], w1_ref[D:, :], preferred_element_type=jnp.float32)
    qfull = jnp.concatenate(
        [jnp.broadcast_to(qc[b:b + 1, :], (NC, qc.shape[1])) for b in range(B)],
        axis=0)
    hmid = jnp.maximum(h1 + qfull + b1_ref[...], 0.0)
    score = jnp.dot(hmid, w2_ref[...], preferred_element_type=jnp.float32)
    o_ref[...] = jnp.broadcast_to(score, o_ref.shape)


@jax.jit
def kernel(r_idx, entities_idx, arity, edge_list, rel_list, query_emb, pos_emb,
           rel_embs, Ws, bs, ln_scales, ln_biases, mlpW1, mlpb1, mlpW2, mlpb2):
    B = r_idx.shape[0]
    E = edge_list.shape[0]
    NL = rel_embs.shape[0]

    all_idx = jnp.transpose(entities_idx, (0, 2, 1))
    mask_for_diff = jnp.all(all_idx[:, :, :1] == all_idx, axis=-1)
    pos_idx = jnp.argmax((~mask_for_diff).astype(jnp.int32), axis=1)
    query = query_emb[r_idx[:, 0]]
    rng = jnp.arange(A)[None, :]
    result = ((rng < arity[:, :1]) & (rng != pos_idx[:, None])).astype(jnp.int32)
    tgt = all_idx[:, :, 0] * result
    vals = query[:, None, :] + pos_emb[result * jnp.arange(1, A + 1)[None, :]]
    vals = vals * (tgt != 0)[:, :, None].astype(jnp.float32)
    collapsed = jnp.take_along_axis(
        all_idx,
        jnp.broadcast_to(pos_idx[:, None, None], (B, 1, all_idx.shape[2])),
        axis=1)[:, 0, :]

    edgeT = jnp.zeros((A, EP), jnp.int32).at[:, :E].set(edge_list.T)
    rel_pad = jnp.zeros((EP,), jnp.int32).at[:E].set(rel_list)
    pos_rows = pos_emb[1:A + 1]

    x = pl.pallas_call(
        _init_body,
        in_specs=[pl.BlockSpec(memory_space=pltpu.SMEM), pl.BlockSpec()],
        out_shape=jax.ShapeDtypeStruct((B, NP, D), jnp.float32),
    )(tgt, vals)

    sc_layer = _make_sc_layer()

    BLK = 256
    nrows = B * NP
    dense = pl.pallas_call(
        _dense_body,
        grid=(nrows // BLK,),
        in_specs=[
            pl.BlockSpec((BLK, D), lambda i: (i, 0)),
            pl.BlockSpec((BLK, D), lambda i: (i, 0)),
            pl.BlockSpec((D, D), lambda i: (0, 0)),
            pl.BlockSpec((1, D), lambda i: (0, 0)),
            pl.BlockSpec((1, D), lambda i: (0, 0)),
            pl.BlockSpec((1, D), lambda i: (0, 0)),
        ],
        out_specs=pl.BlockSpec((BLK, D), lambda i: (i, 0)),
        out_shape=jax.ShapeDtypeStruct((nrows, D), jnp.float32),
    )

    bits = ((jnp.arange(64)[:, None] >> jnp.arange(A)[None, :]) & 1)
    possum = bits.astype(jnp.float32) @ pos_rows
    for l in range(NL):
        agg = sc_layer(x.reshape(B * NP, D), edgeT, rel_pad, rel_embs[l],
                       pos_rows, possum)
        xf = dense(agg.reshape(nrows, D), x.reshape(nrows, D), Ws[l],
                   bs[l][None, :], ln_scales[l][None, :], ln_biases[l][None, :])
        x = xf.reshape(B, NP, D)

    NC = collapsed.shape[1]
    FEAT = mlpW1.shape[0]
    head = pl.pallas_call(
        _head_body,
        in_specs=[pl.BlockSpec(memory_space=pltpu.SMEM)] +
                 [pl.BlockSpec()] * 5,
        out_shape=jax.ShapeDtypeStruct((B * NC, D), jnp.float32),
        scratch_shapes=[pltpu.VMEM((B * NC, D), jnp.float32)],
    )
    sc_out = head(collapsed, x, query, mlpW1, mlpb1[None, :], mlpW2)
    return sc_out[:, 0].reshape(B, NC) + mlpb2[0]


# speedup vs baseline: 7.4351x; 1.0184x over previous
import functools
import jax
import jax.numpy as jnp
from jax import lax
from jax.experimental import pallas as pl
from jax.experimental.pallas import tpu as pltpu
from jax.experimental.pallas import tpu_sc as plsc

D = 128
A = 6
NLANE = 16
NSUB = 16
NCORE = 2
NP = 10112
EP = 20480
CHUNK = 16
RCHUNK = 64
NRCHUNK = NP // RCHUNK


EPT = EP // NSUB
NCH = EPT // CHUNK


def _sc_layer_body(x_hbm, edgeT_hbm, rel_list_hbm, rel_tab_hbm, pos_hbm,
                   possum_hbm, out_hbm, ebuf, gidx, ridx, midx, sidx, xrows,
                   relrows, posrows, msgf, pos_v, agg_sh, sem, sem2):
    c = lax.axis_index("c")
    s = lax.axis_index("s")

    pltpu.sync_copy(pos_hbm, pos_v)

    zvec = jnp.zeros((NLANE,), jnp.float32)

    def zrow(r, carry):
        for d in range(D // NLANE):
            msgf[r, pl.ds(d * NLANE, NLANE)] = zvec
        return carry

    lax.fori_loop(0, RCHUNK, zrow, 0)
    for t in range((NRCHUNK + NSUB - 1) // NSUB):
        cid = s + t * NSUB

        @pl.when(cid < NRCHUNK)
        def _():
            pltpu.sync_copy(msgf.at[pl.ds(0, RCHUNK)],
                            agg_sh.at[pl.ds(cid * RCHUNK, RCHUNK)])
    plsc.subcore_barrier()

    e0 = s * EPT
    xoff = c * NP

    def load_and_fire(k, p):
        off = e0 + k * CHUNK
        for j in range(A):
            pltpu.sync_copy(edgeT_hbm.at[j, pl.ds(off, CHUNK)], ebuf.at[p, j])
        pltpu.sync_copy(rel_list_hbm.at[pl.ds(off, CHUNK)], ridx.at[p])
        sl = pl.ds(0, NLANE)
        mc = jnp.zeros((NLANE,), jnp.int32)
        for j in range(A):
            ej = ebuf[p, j, sl]
            gidx[p, j, sl] = ej + jnp.full((NLANE,), xoff, jnp.int32)
            mc = mc + jnp.where(ej != 0, 1 << j, 0)
        midx[p, sl] = mc
        for j in range(A):
            pltpu.async_copy(x_hbm.at[gidx.at[p, j]], xrows.at[p, j], sem)

    def drain_x(p):
        for j in range(A):
            pltpu.make_async_copy(x_hbm.at[gidx.at[p, j]], xrows.at[p, j],
                                  sem).wait()

    load_and_fire(0, 0)

    def pair_body(t, carry):
        for p in (0, 1):
            k = 2 * t + p
            drain_x(p)
            pltpu.async_copy(rel_tab_hbm.at[ridx.at[p]], relrows, sem2)
            pltpu.async_copy(possum_hbm.at[midx.at[p]], posrows, sem2)

            @pl.when(k + 1 < NCH)
            def _():
                load_and_fire(k + 1, 1 - p)

            pltpu.make_async_copy(rel_tab_hbm.at[ridx.at[p]], relrows,
                                  sem2).wait()
            pltpu.make_async_copy(possum_hbm.at[midx.at[p]], posrows,
                                  sem2).wait()

            def edge_body(e, carry2):
                srow = []
                rrow = []
                for d in range(D // NLANE):
                    sl = pl.ds(d * NLANE, NLANE)
                    acc = posrows[e, sl]
                    for j in range(A):
                        acc = acc + xrows[p, j, e, sl]
                    srow.append(acc)
                    rrow.append(relrows[e, sl])
                for i in range(A):
                    for d in range(D // NLANE):
                        sl = pl.ds(d * NLANE, NLANE)
                        tmp = srow[d] - xrows[p, i, e, sl] - pos_v[i, sl]
                        msgf[i * CHUNK + e, sl] = tmp * rrow[d]
                return carry2

            lax.fori_loop(0, CHUNK, edge_body, 0)
            sl = pl.ds(0, NLANE)
            for i in range(A):
                sidx[pl.ds(i * CHUNK, NLANE)] = ebuf[p, i, sl]
            pltpu.sync_copy(msgf, agg_sh.at[sidx], add=True)
        return carry

    lax.fori_loop(0, NCH // 2, pair_body, 0)
    plsc.subcore_barrier()

    for t in range((NRCHUNK + NSUB - 1) // NSUB):
        cid = s + t * NSUB

        @pl.when(cid < NRCHUNK)
        def _():
            rr = pl.ds(cid * RCHUNK, RCHUNK)
            pltpu.sync_copy(agg_sh.at[rr], msgf.at[pl.ds(0, RCHUNK)])
            pltpu.sync_copy(msgf.at[pl.ds(0, RCHUNK)], out_hbm.at[c].at[rr])


def _make_sc_layer():
    mesh = plsc.VectorSubcoreMesh(core_axis_name="c", subcore_axis_name="s")
    return pl.kernel(
        _sc_layer_body,
        mesh=mesh,
        out_type=jax.ShapeDtypeStruct((NCORE, NP, D), jnp.float32),
        scratch_types=[
            pltpu.VMEM((2, A, CHUNK), jnp.int32),
            pltpu.VMEM((2, A, CHUNK), jnp.int32),
            pltpu.VMEM((2, CHUNK), jnp.int32),
            pltpu.VMEM((2, CHUNK), jnp.int32),
            pltpu.VMEM((A * CHUNK,), jnp.int32),
            pltpu.VMEM((2, A, CHUNK, D), jnp.float32),
            pltpu.VMEM((CHUNK, D), jnp.float32),
            pltpu.VMEM((CHUNK, D), jnp.float32),
            pltpu.VMEM((A * CHUNK, D), jnp.float32),
            pltpu.VMEM((A, D), jnp.float32),
            pltpu.VMEM_SHARED((NP, D), jnp.float32),
            pltpu.SemaphoreType.DMA,
            pltpu.SemaphoreType.DMA,
        ],
    )


def _init_body(tgt_ref, vals_ref, o_ref):
    o_ref[...] = jnp.zeros(o_ref.shape, o_ref.dtype)
    B = tgt_ref.shape[0]
    for b in range(B):
        for j in range(A):
            t = tgt_ref[b, j]
            cur = o_ref[b, pl.ds(t, 1), :]
            o_ref[b, pl.ds(t, 1), :] = cur + vals_ref[b, j, :][None, :]


def _dense_body(agg_ref, xp_ref, w_ref, b_ref, sc_ref, bi_ref, o_ref):
    h = jnp.dot(agg_ref[...], w_ref[...],
                preferred_element_type=jnp.float32) + b_ref[...]
    mu = jnp.mean(h, axis=-1, keepdims=True)
    hc = h - mu
    var = jnp.mean(hc * hc, axis=-1, keepdims=True)
    h = hc * lax.rsqrt(var + 1e-5) * sc_ref[...] + bi_ref[...]
    blk = agg_ref.shape[0]
    rowid = (lax.broadcasted_iota(jnp.int32, (blk, 1), 0)
             + pl.program_id(0) * blk)
    keep = ((rowid % NP) != 0).astype(jnp.float32)
    o_ref[...] = (jnp.maximum(h, 0.0) + xp_ref[...]) * keep


def _head_body(coll_ref, x_ref, q_ref, w1_ref, b1_ref, w2_ref, o_ref, xg_ref):
    B, NC = coll_ref.shape
    for b in range(B):
        for n in range(NC):
            t = coll_ref[b, n]
            xg_ref[pl.ds(b * NC + n, 1), :] = x_ref[b, pl.ds(t, 1), :]
    h1 = jnp.dot(xg_ref[...], w1_ref[:D, :], preferred_element_type=jnp.float32)
    qc = jnp.dot(q_ref[...], w1_ref[D:, :], preferred_element_type=jnp.float32)
    qfull = jnp.concatenate(
        [jnp.broadcast_to(qc[b:b + 1, :], (NC, qc.shape[1])) for b in range(B)],
        axis=0)
    hmid = jnp.maximum(h1 + qfull + b1_ref[...], 0.0)
    score = jnp.dot(hmid, w2_ref[...], preferred_element_type=jnp.float32)
    o_ref[...] = jnp.broadcast_to(score, o_ref.shape)


@jax.jit
def kernel(r_idx, entities_idx, arity, edge_list, rel_list, query_emb, pos_emb,
           rel_embs, Ws, bs, ln_scales, ln_biases, mlpW1, mlpb1, mlpW2, mlpb2):
    B = r_idx.shape[0]
    E = edge_list.shape[0]
    NL = rel_embs.shape[0]

    all_idx = jnp.transpose(entities_idx, (0, 2, 1))
    mask_for_diff = jnp.all(all_idx[:, :, :1] == all_idx, axis=-1)
    pos_idx = jnp.argmax((~mask_for_diff).astype(jnp.int32), axis=1)
    query = query_emb[r_idx[:, 0]]
    rng = jnp.arange(A)[None, :]
    result = ((rng < arity[:, :1]) & (rng != pos_idx[:, None])).astype(jnp.int32)
    tgt = all_idx[:, :, 0] * result
    vals = query[:, None, :] + pos_emb[result * jnp.arange(1, A + 1)[None, :]]
    vals = vals * (tgt != 0)[:, :, None].astype(jnp.float32)
    collapsed = jnp.take_along_axis(
        all_idx,
        jnp.broadcast_to(pos_idx[:, None, None], (B, 1, all_idx.shape[2])),
        axis=1)[:, 0, :]

    edgeT = jnp.zeros((A, EP), jnp.int32).at[:, :E].set(edge_list.T)
    rel_pad = jnp.zeros((EP,), jnp.int32).at[:E].set(rel_list)
    pos_rows = pos_emb[1:A + 1]

    x = pl.pallas_call(
        _init_body,
        in_specs=[pl.BlockSpec(memory_space=pltpu.SMEM), pl.BlockSpec()],
        out_shape=jax.ShapeDtypeStruct((B, NP, D), jnp.float32),
    )(tgt, vals)

    sc_layer = _make_sc_layer()

    BLK = 256
    nrows = B * NP
    dense = pl.pallas_call(
        _dense_body,
        grid=(nrows // BLK,),
        in_specs=[
            pl.BlockSpec((BLK, D), lambda i: (i, 0)),
            pl.BlockSpec((BLK, D), lambda i: (i, 0)),
            pl.BlockSpec((D, D), lambda i: (0, 0)),
            pl.BlockSpec((1, D), lambda i: (0, 0)),
            pl.BlockSpec((1, D), lambda i: (0, 0)),
            pl.BlockSpec((1, D), lambda i: (0, 0)),
        ],
        out_specs=pl.BlockSpec((BLK, D), lambda i: (i, 0)),
        out_shape=jax.ShapeDtypeStruct((nrows, D), jnp.float32),
    )

    bits = ((jnp.arange(64)[:, None] >> jnp.arange(A)[None, :]) & 1)
    possum = bits.astype(jnp.float32) @ pos_rows
    for l in range(NL):
        agg = sc_layer(x.reshape(B * NP, D), edgeT, rel_pad, rel_embs[l],
                       pos_rows, possum)
        xf = dense(agg.reshape(nrows, D), x.reshape(nrows, D), Ws[l],
                   bs[l][None, :], ln_scales[l][None, :], ln_biases[l][None, :])
        x = xf.reshape(B, NP, D)

    NC = collapsed.shape[1]
    FEAT = mlpW1.shape[0]
    head = pl.pallas_call(
        _head_body,
        in_specs=[pl.BlockSpec(memory_space=pltpu.SMEM)] +
                 [pl.BlockSpec()] * 5,
        out_shape=jax.ShapeDtypeStruct((B * NC, D), jnp.float32),
        scratch_shapes=[pltpu.VMEM((B * NC, D), jnp.float32)],
    )
    sc_out = head(collapsed, x, query, mlpW1, mlpb1[None, :], mlpW2)
    return sc_out[:, 0].reshape(B, NC) + mlpb2[0]


# speedup vs baseline: 7.4671x; 1.0043x over previous
import functools
import jax
import jax.numpy as jnp
from jax import lax
from jax.experimental import pallas as pl
from jax.experimental.pallas import tpu as pltpu
from jax.experimental.pallas import tpu_sc as plsc

D = 128
A = 6
NLANE = 16
NSUB = 16
NCORE = 2
NP = 10112
EP = 20480
CHUNK = 16
RCHUNK = 64
NRCHUNK = NP // RCHUNK


EPT = EP // NSUB
NCH = EPT // CHUNK


def _sc_layer_body(x_hbm, edgeT_hbm, rel_list_hbm, rel_tab_hbm, pos_hbm,
                   possum_hbm, out_hbm, ebuf, gidx, ridx, midx, sidx, xrows,
                   relrows, posrows, msgf, pos_v, agg_sh, xsema, xsemb, sem2,
                   sem3):
    xsems = (xsema, xsemb)
    c = lax.axis_index("c")
    s = lax.axis_index("s")

    pltpu.sync_copy(pos_hbm, pos_v)

    zvec = jnp.zeros((NLANE,), jnp.float32)

    def zrow(r, carry):
        for d in range(D // NLANE):
            msgf[r, pl.ds(d * NLANE, NLANE)] = zvec
        return carry

    lax.fori_loop(0, RCHUNK, zrow, 0)
    for t in range((NRCHUNK + NSUB - 1) // NSUB):
        cid = s + t * NSUB

        @pl.when(cid < NRCHUNK)
        def _():
            pltpu.sync_copy(msgf.at[pl.ds(0, RCHUNK)],
                            agg_sh.at[pl.ds(cid * RCHUNK, RCHUNK)])
    plsc.subcore_barrier()

    e0 = s * EPT
    xoff = c * NP

    sl16 = pl.ds(0, NLANE)

    def build_and_fire_x(p):
        mc = jnp.zeros((NLANE,), jnp.int32)
        for j in range(A):
            ej = ebuf[p, j, sl16]
            gidx[p, j, sl16] = ej + jnp.full((NLANE,), xoff, jnp.int32)
            mc = mc + jnp.where(ej != 0, 1 << j, 0)
        midx[p, sl16] = mc
        for j in range(A):
            pltpu.async_copy(x_hbm.at[gidx.at[p, j]], xrows.at[p, j],
                             xsems[p])

    for j in range(A):
        pltpu.sync_copy(edgeT_hbm.at[j, pl.ds(e0, CHUNK)], ebuf.at[0, j])
    pltpu.sync_copy(rel_list_hbm.at[pl.ds(e0, CHUNK)], ridx.at[0])
    build_and_fire_x(0)
    off1 = e0 + CHUNK
    for j in range(A):
        pltpu.async_copy(edgeT_hbm.at[j, pl.ds(off1, CHUNK)], ebuf.at[1, j],
                         sem3)
    pltpu.async_copy(rel_list_hbm.at[pl.ds(off1, CHUNK)], ridx.at[1], sem3)

    def pair_body(t, carry):
        for p in (0, 1):
            q = 1 - p
            k = 2 * t + p
            pltpu.async_copy(rel_tab_hbm.at[ridx.at[p]], relrows, sem2)
            pltpu.async_copy(possum_hbm.at[midx.at[p]], posrows, sem2)

            @pl.when(k + 1 < NCH)
            def _():
                offn = e0 + (k + 1) * CHUNK
                for j in range(A):
                    pltpu.make_async_copy(
                        edgeT_hbm.at[j, pl.ds(offn, CHUNK)], ebuf.at[q, j],
                        sem3).wait()
                pltpu.make_async_copy(rel_list_hbm.at[pl.ds(offn, CHUNK)],
                                      ridx.at[q], sem3).wait()
                build_and_fire_x(q)

            for i in range(A):
                sidx[pl.ds(i * CHUNK, NLANE)] = ebuf[p, i, sl16]
            pltpu.make_async_copy(rel_tab_hbm.at[ridx.at[p]], relrows,
                                  sem2).wait()
            pltpu.make_async_copy(possum_hbm.at[midx.at[p]], posrows,
                                  sem2).wait()

            @pl.when(k + 2 < NCH)
            def _():
                off2 = e0 + (k + 2) * CHUNK
                for j in range(A):
                    pltpu.async_copy(edgeT_hbm.at[j, pl.ds(off2, CHUNK)],
                                     ebuf.at[p, j], sem3)
                pltpu.async_copy(rel_list_hbm.at[pl.ds(off2, CHUNK)],
                                 ridx.at[p], sem3)

            for j in range(A):
                pltpu.make_async_copy(x_hbm.at[gidx.at[p, j]],
                                      xrows.at[p, j], xsems[p]).wait()

            def edge_body(e, carry2):
                srow = []
                rrow = []
                for d in range(D // NLANE):
                    sl = pl.ds(d * NLANE, NLANE)
                    acc = posrows[e, sl]
                    for j in range(A):
                        acc = acc + xrows[p, j, e, sl]
                    srow.append(acc)
                    rrow.append(relrows[e, sl])
                for i in range(A):
                    for d in range(D // NLANE):
                        sl = pl.ds(d * NLANE, NLANE)
                        tmp = srow[d] - xrows[p, i, e, sl] - pos_v[i, sl]
                        msgf[i * CHUNK + e, sl] = tmp * rrow[d]
                return carry2

            lax.fori_loop(0, CHUNK, edge_body, 0)
            pltpu.sync_copy(msgf, agg_sh.at[sidx], add=True)
        return carry

    lax.fori_loop(0, NCH // 2, pair_body, 0)
    plsc.subcore_barrier()

    for t in range((NRCHUNK + NSUB - 1) // NSUB):
        cid = s + t * NSUB

        @pl.when(cid < NRCHUNK)
        def _():
            rr = pl.ds(cid * RCHUNK, RCHUNK)
            pltpu.sync_copy(agg_sh.at[rr], msgf.at[pl.ds(0, RCHUNK)])
            pltpu.sync_copy(msgf.at[pl.ds(0, RCHUNK)], out_hbm.at[c].at[rr])


def _make_sc_layer():
    mesh = plsc.VectorSubcoreMesh(core_axis_name="c", subcore_axis_name="s")
    return pl.kernel(
        _sc_layer_body,
        mesh=mesh,
        out_type=jax.ShapeDtypeStruct((NCORE, NP, D), jnp.float32),
        scratch_types=[
            pltpu.VMEM((2, A, CHUNK), jnp.int32),
            pltpu.VMEM((2, A, CHUNK), jnp.int32),
            pltpu.VMEM((2, CHUNK), jnp.int32),
            pltpu.VMEM((2, CHUNK), jnp.int32),
            pltpu.VMEM((A * CHUNK,), jnp.int32),
            pltpu.VMEM((2, A, CHUNK, D), jnp.float32),
            pltpu.VMEM((CHUNK, D), jnp.float32),
            pltpu.VMEM((CHUNK, D), jnp.float32),
            pltpu.VMEM((A * CHUNK, D), jnp.float32),
            pltpu.VMEM((A, D), jnp.float32),
            pltpu.VMEM_SHARED((NP, D), jnp.float32),
            pltpu.SemaphoreType.DMA,
            pltpu.SemaphoreType.DMA,
            pltpu.SemaphoreType.DMA,
            pltpu.SemaphoreType.DMA,
        ],
    )


def _init_body(tgt_ref, vals_ref, o_ref):
    o_ref[...] = jnp.zeros(o_ref.shape, o_ref.dtype)
    B = tgt_ref.shape[0]
    for b in range(B):
        for j in range(A):
            t = tgt_ref[b, j]
            cur = o_ref[b, pl.ds(t, 1), :]
            o_ref[b, pl.ds(t, 1), :] = cur + vals_ref[b, j, :][None, :]


def _dense_body(agg_ref, xp_ref, w_ref, b_ref, sc_ref, bi_ref, o_ref):
    h = jnp.dot(agg_ref[...], w_ref[...],
                preferred_element_type=jnp.float32) + b_ref[...]
    mu = jnp.mean(h, axis=-1, keepdims=True)
    hc = h - mu
    var = jnp.mean(hc * hc, axis=-1, keepdims=True)
    h = hc * lax.rsqrt(var + 1e-5) * sc_ref[...] + bi_ref[...]
    blk = agg_ref.shape[0]
    rowid = (lax.broadcasted_iota(jnp.int32, (blk, 1), 0)
             + pl.program_id(0) * blk)
    keep = ((rowid % NP) != 0).astype(jnp.float32)
    o_ref[...] = (jnp.maximum(h, 0.0) + xp_ref[...]) * keep


def _head_body(coll_ref, x_ref, q_ref, w1_ref, b1_ref, w2_ref, o_ref, xg_ref):
    B, NC = coll_ref.shape
    for b in range(B):
        for n in range(NC):
            t = coll_ref[b, n]
            xg_ref[pl.ds(b * NC + n, 1), :] = x_ref[b, pl.ds(t, 1), :]
    h1 = jnp.dot(xg_ref[...], w1_ref[:D, :], preferred_element_type=jnp.float32)
    qc = jnp.dot(q_ref[...], w1_ref[D:, :], preferred_element_type=jnp.float32)
    qfull = jnp.concatenate(
        [jnp.broadcast_to(qc[b:b + 1, :], (NC, qc.shape[1])) for b in range(B)],
        axis=0)
    hmid = jnp.maximum(h1 + qfull + b1_ref[...], 0.0)
    score = jnp.dot(hmid, w2_ref[...], preferred_element_type=jnp.float32)
    o_ref[...] = jnp.broadcast_to(score, o_ref.shape)


@jax.jit
def kernel(r_idx, entities_idx, arity, edge_list, rel_list, query_emb, pos_emb,
           rel_embs, Ws, bs, ln_scales, ln_biases, mlpW1, mlpb1, mlpW2, mlpb2):
    B = r_idx.shape[0]
    E = edge_list.shape[0]
    NL = rel_embs.shape[0]

    all_idx = jnp.transpose(entities_idx, (0, 2, 1))
    mask_for_diff = jnp.all(all_idx[:, :, :1] == all_idx, axis=-1)
    pos_idx = jnp.argmax((~mask_for_diff).astype(jnp.int32), axis=1)
    query = query_emb[r_idx[:, 0]]
    rng = jnp.arange(A)[None, :]
    result = ((rng < arity[:, :1]) & (rng != pos_idx[:, None])).astype(jnp.int32)
    tgt = all_idx[:, :, 0] * result
    vals = query[:, None, :] + pos_emb[result * jnp.arange(1, A + 1)[None, :]]
    vals = vals * (tgt != 0)[:, :, None].astype(jnp.float32)
    collapsed = jnp.take_along_axis(
        all_idx,
        jnp.broadcast_to(pos_idx[:, None, None], (B, 1, all_idx.shape[2])),
        axis=1)[:, 0, :]

    edgeT = jnp.zeros((A, EP), jnp.int32).at[:, :E].set(edge_list.T)
    rel_pad = jnp.zeros((EP,), jnp.int32).at[:E].set(rel_list)
    pos_rows = pos_emb[1:A + 1]

    x = pl.pallas_call(
        _init_body,
        in_specs=[pl.BlockSpec(memory_space=pltpu.SMEM), pl.BlockSpec()],
        out_shape=jax.ShapeDtypeStruct((B, NP, D), jnp.float32),
    )(tgt, vals)

    sc_layer = _make_sc_layer()

    BLK = 256
    nrows = B * NP
    dense = pl.pallas_call(
        _dense_body,
        grid=(nrows // BLK,),
        in_specs=[
            pl.BlockSpec((BLK, D), lambda i: (i, 0)),
            pl.BlockSpec((BLK, D), lambda i: (i, 0)),
            pl.BlockSpec((D, D), lambda i: (0, 0)),
            pl.BlockSpec((1, D), lambda i: (0, 0)),
            pl.BlockSpec((1, D), lambda i: (0, 0)),
            pl.BlockSpec((1, D), lambda i: (0, 0)),
        ],
        out_specs=pl.BlockSpec((BLK, D), lambda i: (i, 0)),
        out_shape=jax.ShapeDtypeStruct((nrows, D), jnp.float32),
    )

    bits = ((jnp.arange(64)[:, None] >> jnp.arange(A)[None, :]) & 1)
    possum = bits.astype(jnp.float32) @ pos_rows
    for l in range(NL):
        agg = sc_layer(x.reshape(B * NP, D), edgeT, rel_pad, rel_embs[l],
                       pos_rows, possum)
        xf = dense(agg.reshape(nrows, D), x.reshape(nrows, D), Ws[l],
                   bs[l][None, :], ln_scales[l][None, :], ln_biases[l][None, :])
        x = xf.reshape(B, NP, D)

    NC = collapsed.shape[1]
    FEAT = mlpW1.shape[0]
    head = pl.pallas_call(
        _head_body,
        in_specs=[pl.BlockSpec(memory_space=pltpu.SMEM)] +
                 [pl.BlockSpec()] * 5,
        out_shape=jax.ShapeDtypeStruct((B * NC, D), jnp.float32),
        scratch_shapes=[pltpu.VMEM((B * NC, D), jnp.float32)],
    )
    sc_out = head(collapsed, x, query, mlpW1, mlpb1[None, :], mlpW2)
    return sc_out[:, 0].reshape(B, NC) + mlpb2[0]


# speedup vs baseline: 7.5170x; 1.0067x over previous
import functools
import jax
import jax.numpy as jnp
from jax import lax
from jax.experimental import pallas as pl
from jax.experimental.pallas import tpu as pltpu
from jax.experimental.pallas import tpu_sc as plsc

D = 128
A = 6
NLANE = 16
NSUB = 16
NCORE = 2
NP = 10112
EP = 20480
CHUNK = 16
RCHUNK = 64
NRCHUNK = NP // RCHUNK


EPT = EP // NSUB
NCH = EPT // CHUNK


def _sc_layer_body(x_hbm, edgeT_hbm, rel_list_hbm, rel_tab_hbm, pos_hbm,
                   possum_hbm, out_hbm, ebuf, gidx, ridx, midx, sidx, xrows,
                   relrows, posrows, msgf, pos_v, agg_sh, xsema, xsemb, sem2,
                   sem3):
    xsems = (xsema, xsemb)
    c = lax.axis_index("c")
    s = lax.axis_index("s")

    pltpu.sync_copy(pos_hbm, pos_v)

    zvec = jnp.zeros((NLANE,), jnp.float32)

    def zrow(r, carry):
        for d in range(D // NLANE):
            msgf[r, pl.ds(d * NLANE, NLANE)] = zvec
        return carry

    lax.fori_loop(0, RCHUNK, zrow, 0)
    for t in range((NRCHUNK + NSUB - 1) // NSUB):
        cid = s + t * NSUB

        @pl.when(cid < NRCHUNK)
        def _():
            pltpu.sync_copy(msgf.at[pl.ds(0, RCHUNK)],
                            agg_sh.at[pl.ds(cid * RCHUNK, RCHUNK)])
    plsc.subcore_barrier()

    e0 = s * EPT
    xoff = c * NP

    sl16 = pl.ds(0, NLANE)

    def build_and_fire_x(p):
        mc = jnp.zeros((NLANE,), jnp.int32)
        for j in range(A):
            ej = ebuf[p, j, sl16]
            gidx[p, j, sl16] = ej + jnp.full((NLANE,), xoff, jnp.int32)
            mc = mc + jnp.where(ej != 0, 1 << j, 0)
        midx[p, sl16] = mc
        for j in range(A):
            pltpu.async_copy(x_hbm.at[gidx.at[p, j]], xrows.at[p, j],
                             xsems[p])

    for j in range(A):
        pltpu.sync_copy(edgeT_hbm.at[j, pl.ds(e0, CHUNK)], ebuf.at[0, j])
    pltpu.sync_copy(rel_list_hbm.at[pl.ds(e0, CHUNK)], ridx.at[0])
    build_and_fire_x(0)
    off1 = e0 + CHUNK
    for j in range(A):
        pltpu.async_copy(edgeT_hbm.at[j, pl.ds(off1, CHUNK)], ebuf.at[1, j],
                         sem3)
    pltpu.async_copy(rel_list_hbm.at[pl.ds(off1, CHUNK)], ridx.at[1], sem3)

    def pair_body(t, carry):
        for p in (0, 1):
            q = 1 - p
            k = 2 * t + p
            pltpu.async_copy(rel_tab_hbm.at[ridx.at[p]], relrows, sem2)
            pltpu.async_copy(possum_hbm.at[midx.at[p]], posrows, sem2)

            @pl.when(k + 1 < NCH)
            def _():
                offn = e0 + (k + 1) * CHUNK
                for j in range(A):
                    pltpu.make_async_copy(
                        edgeT_hbm.at[j, pl.ds(offn, CHUNK)], ebuf.at[q, j],
                        sem3).wait()
                pltpu.make_async_copy(rel_list_hbm.at[pl.ds(offn, CHUNK)],
                                      ridx.at[q], sem3).wait()
                build_and_fire_x(q)

            for i in range(A):
                sidx[pl.ds(i * CHUNK, NLANE)] = ebuf[p, i, sl16]
            pltpu.make_async_copy(rel_tab_hbm.at[ridx.at[p]], relrows,
                                  sem2).wait()
            pltpu.make_async_copy(possum_hbm.at[midx.at[p]], posrows,
                                  sem2).wait()

            @pl.when(k + 2 < NCH)
            def _():
                off2 = e0 + (k + 2) * CHUNK
                for j in range(A):
                    pltpu.async_copy(edgeT_hbm.at[j, pl.ds(off2, CHUNK)],
                                     ebuf.at[p, j], sem3)
                pltpu.async_copy(rel_list_hbm.at[pl.ds(off2, CHUNK)],
                                 ridx.at[p], sem3)

            for j in range(A):
                pltpu.make_async_copy(x_hbm.at[gidx.at[p, j]],
                                      xrows.at[p, j], xsems[p]).wait()

            @plsc.parallel_loop(0, CHUNK, unroll=4)
            def _(e):
                for d in range(D // NLANE):
                    sl = pl.ds(d * NLANE, NLANE)
                    xv = [xrows[p, j, e, sl] for j in range(A)]
                    sd = posrows[e, sl]
                    for j in range(A):
                        sd = sd + xv[j]
                    rd = relrows[e, sl]
                    for i in range(A):
                        msgf[i * CHUNK + e, sl] = (sd - xv[i]
                                                   - pos_v[i, sl]) * rd
            pltpu.sync_copy(msgf, agg_sh.at[sidx], add=True)
        return carry

    lax.fori_loop(0, NCH // 2, pair_body, 0)
    plsc.subcore_barrier()

    for t in range((NRCHUNK + NSUB - 1) // NSUB):
        cid = s + t * NSUB

        @pl.when(cid < NRCHUNK)
        def _():
            rr = pl.ds(cid * RCHUNK, RCHUNK)
            pltpu.sync_copy(agg_sh.at[rr], msgf.at[pl.ds(0, RCHUNK)])
            pltpu.sync_copy(msgf.at[pl.ds(0, RCHUNK)], out_hbm.at[c].at[rr])


def _make_sc_layer():
    mesh = plsc.VectorSubcoreMesh(core_axis_name="c", subcore_axis_name="s")
    return pl.kernel(
        _sc_layer_body,
        mesh=mesh,
        out_type=jax.ShapeDtypeStruct((NCORE, NP, D), jnp.float32),
        scratch_types=[
            pltpu.VMEM((2, A, CHUNK), jnp.int32),
            pltpu.VMEM((2, A, CHUNK), jnp.int32),
            pltpu.VMEM((2, CHUNK), jnp.int32),
            pltpu.VMEM((2, CHUNK), jnp.int32),
            pltpu.VMEM((A * CHUNK,), jnp.int32),
            pltpu.VMEM((2, A, CHUNK, D), jnp.float32),
            pltpu.VMEM((CHUNK, D), jnp.float32),
            pltpu.VMEM((CHUNK, D), jnp.float32),
            pltpu.VMEM((A * CHUNK, D), jnp.float32),
            pltpu.VMEM((A, D), jnp.float32),
            pltpu.VMEM_SHARED((NP, D), jnp.float32),
            pltpu.SemaphoreType.DMA,
            pltpu.SemaphoreType.DMA,
            pltpu.SemaphoreType.DMA,
            pltpu.SemaphoreType.DMA,
        ],
    )


def _init_body(tgt_ref, vals_ref, o_ref):
    o_ref[...] = jnp.zeros(o_ref.shape, o_ref.dtype)
    B = tgt_ref.shape[0]
    for b in range(B):
        for j in range(A):
            t = tgt_ref[b, j]
            cur = o_ref[b, pl.ds(t, 1), :]
            o_ref[b, pl.ds(t, 1), :] = cur + vals_ref[b, j, :][None, :]


def _dense_body(agg_ref, xp_ref, w_ref, b_ref, sc_ref, bi_ref, o_ref):
    h = jnp.dot(agg_ref[...], w_ref[...],
                preferred_element_type=jnp.float32) + b_ref[...]
    mu = jnp.mean(h, axis=-1, keepdims=True)
    hc = h - mu
    var = jnp.mean(hc * hc, axis=-1, keepdims=True)
    h = hc * lax.rsqrt(var + 1e-5) * sc_ref[...] + bi_ref[...]
    blk = agg_ref.shape[0]
    rowid = (lax.broadcasted_iota(jnp.int32, (blk, 1), 0)
             + pl.program_id(0) * blk)
    keep = ((rowid % NP) != 0).astype(jnp.float32)
    o_ref[...] = (jnp.maximum(h, 0.0) + xp_ref[...]) * keep


def _head_body(coll_ref, x_ref, q_ref, w1_ref, b1_ref, w2_ref, o_ref, xg_ref):
    B, NC = coll_ref.shape
    for b in range(B):
        for n in range(NC):
            t = coll_ref[b, n]
            xg_ref[pl.ds(b * NC + n, 1), :] = x_ref[b, pl.ds(t, 1), :]
    h1 = jnp.dot(xg_ref[...], w1_ref[:D, :], preferred_element_type=jnp.float32)
    qc = jnp.dot(q_ref[...], w1_ref[D:, :], preferred_element_type=jnp.float32)
    qfull = jnp.concatenate(
        [jnp.broadcast_to(qc[b:b + 1, :], (NC, qc.shape[1])) for b in range(B)],
        axis=0)
    hmid = jnp.maximum(h1 + qfull + b1_ref[...], 0.0)
    score = jnp.dot(hmid, w2_ref[...], preferred_element_type=jnp.float32)
    o_ref[...] = jnp.broadcast_to(score, o_ref.shape)


@jax.jit
def kernel(r_idx, entities_idx, arity, edge_list, rel_list, query_emb, pos_emb,
           rel_embs, Ws, bs, ln_scales, ln_biases, mlpW1, mlpb1, mlpW2, mlpb2):
    B = r_idx.shape[0]
    E = edge_list.shape[0]
    NL = rel_embs.shape[0]

    all_idx = jnp.transpose(entities_idx, (0, 2, 1))
    mask_for_diff = jnp.all(all_idx[:, :, :1] == all_idx, axis=-1)
    pos_idx = jnp.argmax((~mask_for_diff).astype(jnp.int32), axis=1)
    query = query_emb[r_idx[:, 0]]
    rng = jnp.arange(A)[None, :]
    result = ((rng < arity[:, :1]) & (rng != pos_idx[:, None])).astype(jnp.int32)
    tgt = all_idx[:, :, 0] * result
    vals = query[:, None, :] + pos_emb[result * jnp.arange(1, A + 1)[None, :]]
    vals = vals * (tgt != 0)[:, :, None].astype(jnp.float32)
    collapsed = jnp.take_along_axis(
        all_idx,
        jnp.broadcast_to(pos_idx[:, None, None], (B, 1, all_idx.shape[2])),
        axis=1)[:, 0, :]

    edgeT = jnp.zeros((A, EP), jnp.int32).at[:, :E].set(edge_list.T)
    rel_pad = jnp.zeros((EP,), jnp.int32).at[:E].set(rel_list)
    pos_rows = pos_emb[1:A + 1]

    x = pl.pallas_call(
        _init_body,
        in_specs=[pl.BlockSpec(memory_space=pltpu.SMEM), pl.BlockSpec()],
        out_shape=jax.ShapeDtypeStruct((B, NP, D), jnp.float32),
    )(tgt, vals)

    sc_layer = _make_sc_layer()

    BLK = 256
    nrows = B * NP
    dense = pl.pallas_call(
        _dense_body,
        grid=(nrows // BLK,),
        in_specs=[
            pl.BlockSpec((BLK, D), lambda i: (i, 0)),
            pl.BlockSpec((BLK, D), lambda i: (i, 0)),
            pl.BlockSpec((D, D), lambda i: (0, 0)),
            pl.BlockSpec((1, D), lambda i: (0, 0)),
            pl.BlockSpec((1, D), lambda i: (0, 0)),
            pl.BlockSpec((1, D), lambda i: (0, 0)),
        ],
        out_specs=pl.BlockSpec((BLK, D), lambda i: (i, 0)),
        out_shape=jax.ShapeDtypeStruct((nrows, D), jnp.float32),
    )

    bits = ((jnp.arange(64)[:, None] >> jnp.arange(A)[None, :]) & 1)
    possum = bits.astype(jnp.float32) @ pos_rows
    for l in range(NL):
        agg = sc_layer(x.reshape(B * NP, D), edgeT, rel_pad, rel_embs[l],
                       pos_rows, possum)
        xf = dense(agg.reshape(nrows, D), x.reshape(nrows, D), Ws[l],
                   bs[l][None, :], ln_scales[l][None, :], ln_biases[l][None, :])
        x = xf.reshape(B, NP, D)

    NC = collapsed.shape[1]
    FEAT = mlpW1.shape[0]
    head = pl.pallas_call(
        _head_body,
        in_specs=[pl.BlockSpec(memory_space=pltpu.SMEM)] +
                 [pl.BlockSpec()] * 5,
        out_shape=jax.ShapeDtypeStruct((B * NC, D), jnp.float32),
        scratch_shapes=[pltpu.VMEM((B * NC, D), jnp.float32)],
    )
    sc_out = head(collapsed, x, query, mlpW1, mlpb1[None, :], mlpW2)
    return sc_out[:, 0].reshape(B, NC) + mlpb2[0]


# speedup vs baseline: 7.5371x; 1.0027x over previous
import functools
import jax
import jax.numpy as jnp
from jax import lax
from jax.experimental import pallas as pl
from jax.experimental.pallas import tpu as pltpu
from jax.experimental.pallas import tpu_sc as plsc

D = 128
A = 6
NLANE = 16
NSUB = 16
NCORE = 2
NP = 10112
EP = 20480
CHUNK = 16
RCHUNK = 64
NRCHUNK = NP // RCHUNK


EPT = EP // NSUB
NCH = EPT // CHUNK


def _sc_layer_body(x_hbm, edgeT_hbm, rel_list_hbm, rel_tab_hbm, pos_hbm,
                   possum_hbm, out_hbm, ebuf, gidx, ridx, midx, sidx, xrows,
                   relrows, posrows, msgf, pos_v, agg_sh, xsema, xsemb, sem2,
                   sem3, ssem0, ssem1):
    xsems = (xsema, xsemb)
    ssems = (ssem0, ssem1)
    HROWS = (A // 2) * CHUNK
    c = lax.axis_index("c")
    s = lax.axis_index("s")

    pltpu.sync_copy(pos_hbm, pos_v)

    zvec = jnp.zeros((NLANE,), jnp.float32)

    def zrow(r, carry):
        for d in range(D // NLANE):
            msgf[r, pl.ds(d * NLANE, NLANE)] = zvec
        return carry

    lax.fori_loop(0, RCHUNK, zrow, 0)
    for t in range((NRCHUNK + NSUB - 1) // NSUB):
        cid = s + t * NSUB

        @pl.when(cid < NRCHUNK)
        def _():
            pltpu.sync_copy(msgf.at[pl.ds(0, RCHUNK)],
                            agg_sh.at[pl.ds(cid * RCHUNK, RCHUNK)])
    plsc.subcore_barrier()

    e0 = s * EPT
    xoff = c * NP

    sl16 = pl.ds(0, NLANE)

    def build_and_fire_x(p):
        mc = jnp.zeros((NLANE,), jnp.int32)
        for j in range(A):
            ej = ebuf[p, j, sl16]
            gidx[p, j, sl16] = ej + jnp.full((NLANE,), xoff, jnp.int32)
            mc = mc + jnp.where(ej != 0, 1 << j, 0)
        midx[p, sl16] = mc
        for j in range(A):
            pltpu.async_copy(x_hbm.at[gidx.at[p, j]], xrows.at[p, j],
                             xsems[p])

    for j in range(A):
        pltpu.sync_copy(edgeT_hbm.at[j, pl.ds(e0, CHUNK)], ebuf.at[0, j])
    pltpu.sync_copy(rel_list_hbm.at[pl.ds(e0, CHUNK)], ridx.at[0])
    build_and_fire_x(0)
    off1 = e0 + CHUNK
    for j in range(A):
        pltpu.async_copy(edgeT_hbm.at[j, pl.ds(off1, CHUNK)], ebuf.at[1, j],
                         sem3)
    pltpu.async_copy(rel_list_hbm.at[pl.ds(off1, CHUNK)], ridx.at[1], sem3)

    def pair_body(t, carry):
        for p in (0, 1):
            q = 1 - p
            k = 2 * t + p
            pltpu.async_copy(rel_tab_hbm.at[ridx.at[p]], relrows, sem2)
            pltpu.async_copy(possum_hbm.at[midx.at[p]], posrows, sem2)

            @pl.when(k > 0)
            def _():
                for h in (0, 1):
                    pltpu.make_async_copy(msgf.at[pl.ds(h * HROWS, HROWS)],
                                          agg_sh.at[sidx.at[h]],
                                          ssems[h]).wait()
            for h in (0, 1):
                i0 = h * (A // 2)
                for i in range(i0, i0 + A // 2):
                    sidx[h, pl.ds((i - i0) * CHUNK, NLANE)] = ebuf[p, i, sl16]

            @pl.when(k + 1 < NCH)
            def _():
                offn = e0 + (k + 1) * CHUNK
                for j in range(A):
                    pltpu.make_async_copy(
                        edgeT_hbm.at[j, pl.ds(offn, CHUNK)], ebuf.at[q, j],
                        sem3).wait()
                pltpu.make_async_copy(rel_list_hbm.at[pl.ds(offn, CHUNK)],
                                      ridx.at[q], sem3).wait()
                build_and_fire_x(q)

            pltpu.make_async_copy(rel_tab_hbm.at[ridx.at[p]], relrows,
                                  sem2).wait()
            pltpu.make_async_copy(possum_hbm.at[midx.at[p]], posrows,
                                  sem2).wait()

            @pl.when(k + 2 < NCH)
            def _():
                off2 = e0 + (k + 2) * CHUNK
                for j in range(A):
                    pltpu.async_copy(edgeT_hbm.at[j, pl.ds(off2, CHUNK)],
                                     ebuf.at[p, j], sem3)
                pltpu.async_copy(rel_list_hbm.at[pl.ds(off2, CHUNK)],
                                 ridx.at[p], sem3)

            for j in range(A):
                pltpu.make_async_copy(x_hbm.at[gidx.at[p, j]],
                                      xrows.at[p, j], xsems[p]).wait()

            for h in (0, 1):
                i0 = h * (A // 2)

                @plsc.parallel_loop(0, CHUNK, unroll=4)
                def _(e):
                    for d in range(D // NLANE):
                        sl = pl.ds(d * NLANE, NLANE)
                        xv = [xrows[p, j, e, sl] for j in range(A)]
                        sd = posrows[e, sl]
                        for j in range(A):
                            sd = sd + xv[j]
                        rd = relrows[e, sl]
                        for i in range(i0, i0 + A // 2):
                            msgf[i * CHUNK + e, sl] = (sd - xv[i]
                                                       - pos_v[i, sl]) * rd

                pltpu.async_copy(msgf.at[pl.ds(h * HROWS, HROWS)],
                                 agg_sh.at[sidx.at[h]], ssems[h], add=True)
        return carry

    lax.fori_loop(0, NCH // 2, pair_body, 0)
    for h in (0, 1):
        pltpu.make_async_copy(msgf.at[pl.ds(h * HROWS, HROWS)],
                              agg_sh.at[sidx.at[h]], ssems[h]).wait()
    plsc.subcore_barrier()

    for t in range((NRCHUNK + NSUB - 1) // NSUB):
        cid = s + t * NSUB

        @pl.when(cid < NRCHUNK)
        def _():
            rr = pl.ds(cid * RCHUNK, RCHUNK)
            pltpu.sync_copy(agg_sh.at[rr], msgf.at[pl.ds(0, RCHUNK)])
            pltpu.sync_copy(msgf.at[pl.ds(0, RCHUNK)], out_hbm.at[c].at[rr])


def _make_sc_layer():
    mesh = plsc.VectorSubcoreMesh(core_axis_name="c", subcore_axis_name="s")
    return pl.kernel(
        _sc_layer_body,
        mesh=mesh,
        out_type=jax.ShapeDtypeStruct((NCORE, NP, D), jnp.float32),
        scratch_types=[
            pltpu.VMEM((2, A, CHUNK), jnp.int32),
            pltpu.VMEM((2, A, CHUNK), jnp.int32),
            pltpu.VMEM((2, CHUNK), jnp.int32),
            pltpu.VMEM((2, CHUNK), jnp.int32),
            pltpu.VMEM((2, (A // 2) * CHUNK), jnp.int32),
            pltpu.VMEM((2, A, CHUNK, D), jnp.float32),
            pltpu.VMEM((CHUNK, D), jnp.float32),
            pltpu.VMEM((CHUNK, D), jnp.float32),
            pltpu.VMEM((A * CHUNK, D), jnp.float32),
            pltpu.VMEM((A, D), jnp.float32),
            pltpu.VMEM_SHARED((NP, D), jnp.float32),
            pltpu.SemaphoreType.DMA,
            pltpu.SemaphoreType.DMA,
            pltpu.SemaphoreType.DMA,
            pltpu.SemaphoreType.DMA,
            pltpu.SemaphoreType.DMA,
            pltpu.SemaphoreType.DMA,
        ],
    )


def _init_body(tgt_ref, vals_ref, o_ref):
    o_ref[...] = jnp.zeros(o_ref.shape, o_ref.dtype)
    B = tgt_ref.shape[0]
    for b in range(B):
        for j in range(A):
            t = tgt_ref[b, j]
            cur = o_ref[b, pl.ds(t, 1), :]
            o_ref[b, pl.ds(t, 1), :] = cur + vals_ref[b, j, :][None, :]


def _dense_body(agg_ref, xp_ref, w_ref, b_ref, sc_ref, bi_ref, o_ref):
    h = jnp.dot(agg_ref[...], w_ref[...],
                preferred_element_type=jnp.float32) + b_ref[...]
    mu = jnp.mean(h, axis=-1, keepdims=True)
    hc = h - mu
    var = jnp.mean(hc * hc, axis=-1, keepdims=True)
    h = hc * lax.rsqrt(var + 1e-5) * sc_ref[...] + bi_ref[...]
    blk = agg_ref.shape[0]
    rowid = (lax.broadcasted_iota(jnp.int32, (blk, 1), 0)
             + pl.program_id(0) * blk)
    keep = ((rowid % NP) != 0).astype(jnp.float32)
    o_ref[...] = (jnp.maximum(h, 0.0) + xp_ref[...]) * keep


def _head_body(coll_ref, x_ref, q_ref, w1_ref, b1_ref, w2_ref, o_ref, xg_ref):
    B, NC = coll_ref.shape
    for b in range(B):
        for n in range(NC):
            t = coll_ref[b, n]
            xg_ref[pl.ds(b * NC + n, 1), :] = x_ref[b, pl.ds(t, 1), :]
    h1 = jnp.dot(xg_ref[...], w1_ref[:D, :], preferred_element_type=jnp.float32)
    qc = jnp.dot(q_ref[...], w1_ref[D:, :], preferred_element_type=jnp.float32)
    qfull = jnp.concatenate(
        [jnp.broadcast_to(qc[b:b + 1, :], (NC, qc.shape[1])) for b in range(B)],
        axis=0)
    hmid = jnp.maximum(h1 + qfull + b1_ref[...], 0.0)
    score = jnp.dot(hmid, w2_ref[...], preferred_element_type=jnp.float32)
    o_ref[...] = jnp.broadcast_to(score, o_ref.shape)


@jax.jit
def kernel(r_idx, entities_idx, arity, edge_list, rel_list, query_emb, pos_emb,
           rel_embs, Ws, bs, ln_scales, ln_biases, mlpW1, mlpb1, mlpW2, mlpb2):
    B = r_idx.shape[0]
    E = edge_list.shape[0]
    NL = rel_embs.shape[0]

    all_idx = jnp.transpose(entities_idx, (0, 2, 1))
    mask_for_diff = jnp.all(all_idx[:, :, :1] == all_idx, axis=-1)
    pos_idx = jnp.argmax((~mask_for_diff).astype(jnp.int32), axis=1)
    query = query_emb[r_idx[:, 0]]
    rng = jnp.arange(A)[None, :]
    result = ((rng < arity[:, :1]) & (rng != pos_idx[:, None])).astype(jnp.int32)
    tgt = all_idx[:, :, 0] * result
    vals = query[:, None, :] + pos_emb[result * jnp.arange(1, A + 1)[None, :]]
    vals = vals * (tgt != 0)[:, :, None].astype(jnp.float32)
    collapsed = jnp.take_along_axis(
        all_idx,
        jnp.broadcast_to(pos_idx[:, None, None], (B, 1, all_idx.shape[2])),
        axis=1)[:, 0, :]

    edgeT = jnp.zeros((A, EP), jnp.int32).at[:, :E].set(edge_list.T)
    rel_pad = jnp.zeros((EP,), jnp.int32).at[:E].set(rel_list)
    pos_rows = pos_emb[1:A + 1]

    x = pl.pallas_call(
        _init_body,
        in_specs=[pl.BlockSpec(memory_space=pltpu.SMEM), pl.BlockSpec()],
        out_shape=jax.ShapeDtypeStruct((B, NP, D), jnp.float32),
    )(tgt, vals)

    sc_layer = _make_sc_layer()

    BLK = 256
    nrows = B * NP
    dense = pl.pallas_call(
        _dense_body,
        grid=(nrows // BLK,),
        in_specs=[
            pl.BlockSpec((BLK, D), lambda i: (i, 0)),
            pl.BlockSpec((BLK, D), lambda i: (i, 0)),
            pl.BlockSpec((D, D), lambda i: (0, 0)),
            pl.BlockSpec((1, D), lambda i: (0, 0)),
            pl.BlockSpec((1, D), lambda i: (0, 0)),
            pl.BlockSpec((1, D), lambda i: (0, 0)),
        ],
        out_specs=pl.BlockSpec((BLK, D), lambda i: (i, 0)),
        out_shape=jax.ShapeDtypeStruct((nrows, D), jnp.float32),
    )

    bits = ((jnp.arange(64)[:, None] >> jnp.arange(A)[None, :]) & 1)
    possum = bits.astype(jnp.float32) @ pos_rows
    for l in range(NL):
        agg = sc_layer(x.reshape(B * NP, D), edgeT, rel_pad, rel_embs[l],
                       pos_rows, possum)
        xf = dense(agg.reshape(nrows, D), x.reshape(nrows, D), Ws[l],
                   bs[l][None, :], ln_scales[l][None, :], ln_biases[l][None, :])
        x = xf.reshape(B, NP, D)

    NC = collapsed.shape[1]
    FEAT = mlpW1.shape[0]
    head = pl.pallas_call(
        _head_body,
        in_specs=[pl.BlockSpec(memory_space=pltpu.SMEM)] +
                 [pl.BlockSpec()] * 5,
        out_shape=jax.ShapeDtypeStruct((B * NC, D), jnp.float32),
        scratch_shapes=[pltpu.VMEM((B * NC, D), jnp.float32)],
    )
    sc_out = head(collapsed, x, query, mlpW1, mlpb1[None, :], mlpW2)
    return sc_out[:, 0].reshape(B, NC) + mlpb2[0]


# speedup vs baseline: 7.5390x; 1.0002x over previous
import functools
import jax
import jax.numpy as jnp
from jax import lax
from jax.experimental import pallas as pl
from jax.experimental.pallas import tpu as pltpu
from jax.experimental.pallas import tpu_sc as plsc

D = 128
A = 6
NLANE = 16
NSUB = 16
NCORE = 2
NP = 10112
EP = 20480
CHUNK = 16
RCHUNK = 64
NRCHUNK = NP // RCHUNK


EPT = EP // NSUB
NCH = EPT // CHUNK


def _sc_layer_body(x_hbm, edgeT_hbm, rel_list_hbm, rel_tab_hbm, pos_hbm,
                   possum_hbm, out_hbm, ebuf, gidx, ridx, midx, sidx, xrows,
                   relrows, posrows, msgf, pos_v, agg_sh, xsema, xsemb, sem2,
                   sem3, ssem0, ssem1):
    xsems = (xsema, xsemb)
    ssems = (ssem0, ssem1)
    HROWS = (A // 2) * CHUNK
    c = lax.axis_index("c")
    s = lax.axis_index("s")

    pltpu.sync_copy(pos_hbm, pos_v)

    zvec = jnp.zeros((NLANE,), jnp.float32)

    def zrow(r, carry):
        for d in range(D // NLANE):
            msgf[r, pl.ds(d * NLANE, NLANE)] = zvec
        return carry

    lax.fori_loop(0, RCHUNK, zrow, 0)
    for t in range((NRCHUNK + NSUB - 1) // NSUB):
        cid = s + t * NSUB

        @pl.when(cid < NRCHUNK)
        def _():
            pltpu.sync_copy(msgf.at[pl.ds(0, RCHUNK)],
                            agg_sh.at[pl.ds(cid * RCHUNK, RCHUNK)])
    plsc.subcore_barrier()

    e0 = s * EPT
    xoff = c * NP

    sl16 = pl.ds(0, NLANE)

    def build_and_fire_x(p):
        mc = jnp.zeros((NLANE,), jnp.int32)
        for j in range(A):
            ej = ebuf[p, j, sl16]
            gidx[p, pl.ds(j * CHUNK, NLANE)] = ej + jnp.full((NLANE,), xoff,
                                                             jnp.int32)
            mc = mc + jnp.where(ej != 0, 1 << j, 0)
        midx[p, sl16] = mc
        pltpu.async_copy(x_hbm.at[gidx.at[p]], xrows.at[p], xsems[p])

    for j in range(A):
        pltpu.sync_copy(edgeT_hbm.at[j, pl.ds(e0, CHUNK)], ebuf.at[0, j])
    pltpu.sync_copy(rel_list_hbm.at[pl.ds(e0, CHUNK)], ridx.at[0])
    build_and_fire_x(0)
    off1 = e0 + CHUNK
    for j in range(A):
        pltpu.async_copy(edgeT_hbm.at[j, pl.ds(off1, CHUNK)], ebuf.at[1, j],
                         sem3)
    pltpu.async_copy(rel_list_hbm.at[pl.ds(off1, CHUNK)], ridx.at[1], sem3)

    def pair_body(t, carry):
        for p in (0, 1):
            q = 1 - p
            k = 2 * t + p
            pltpu.async_copy(rel_tab_hbm.at[ridx.at[p]], relrows, sem2)
            pltpu.async_copy(possum_hbm.at[midx.at[p]], posrows, sem2)

            @pl.when(k > 0)
            def _():
                for h in (0, 1):
                    pltpu.make_async_copy(msgf.at[pl.ds(h * HROWS, HROWS)],
                                          agg_sh.at[sidx.at[h]],
                                          ssems[h]).wait()
            for h in (0, 1):
                i0 = h * (A // 2)
                for i in range(i0, i0 + A // 2):
                    sidx[h, pl.ds((i - i0) * CHUNK, NLANE)] = ebuf[p, i, sl16]

            @pl.when(k + 1 < NCH)
            def _():
                offn = e0 + (k + 1) * CHUNK
                for j in range(A):
                    pltpu.make_async_copy(
                        edgeT_hbm.at[j, pl.ds(offn, CHUNK)], ebuf.at[q, j],
                        sem3).wait()
                pltpu.make_async_copy(rel_list_hbm.at[pl.ds(offn, CHUNK)],
                                      ridx.at[q], sem3).wait()
                build_and_fire_x(q)

            pltpu.make_async_copy(rel_tab_hbm.at[ridx.at[p]], relrows,
                                  sem2).wait()
            pltpu.make_async_copy(possum_hbm.at[midx.at[p]], posrows,
                                  sem2).wait()

            @pl.when(k + 2 < NCH)
            def _():
                off2 = e0 + (k + 2) * CHUNK
                for j in range(A):
                    pltpu.async_copy(edgeT_hbm.at[j, pl.ds(off2, CHUNK)],
                                     ebuf.at[p, j], sem3)
                pltpu.async_copy(rel_list_hbm.at[pl.ds(off2, CHUNK)],
                                 ridx.at[p], sem3)

            pltpu.make_async_copy(x_hbm.at[gidx.at[p]], xrows.at[p],
                                  xsems[p]).wait()

            for h in (0, 1):
                i0 = h * (A // 2)

                @plsc.parallel_loop(0, CHUNK, unroll=4)
                def _(e):
                    for d in range(D // NLANE):
                        sl = pl.ds(d * NLANE, NLANE)
                        xv = [xrows[p, j * CHUNK + e, sl] for j in range(A)]
                        sd = posrows[e, sl]
                        for j in range(A):
                            sd = sd + xv[j]
                        rd = relrows[e, sl]
                        for i in range(i0, i0 + A // 2):
                            msgf[i * CHUNK + e, sl] = (sd - xv[i]
                                                       - pos_v[i, sl]) * rd

                pltpu.async_copy(msgf.at[pl.ds(h * HROWS, HROWS)],
                                 agg_sh.at[sidx.at[h]], ssems[h], add=True)
        return carry

    lax.fori_loop(0, NCH // 2, pair_body, 0)
    for h in (0, 1):
        pltpu.make_async_copy(msgf.at[pl.ds(h * HROWS, HROWS)],
                              agg_sh.at[sidx.at[h]], ssems[h]).wait()
    plsc.subcore_barrier()

    for t in range((NRCHUNK + NSUB - 1) // NSUB):
        cid = s + t * NSUB

        @pl.when(cid < NRCHUNK)
        def _():
            rr = pl.ds(cid * RCHUNK, RCHUNK)
            pltpu.sync_copy(agg_sh.at[rr], msgf.at[pl.ds(0, RCHUNK)])
            pltpu.sync_copy(msgf.at[pl.ds(0, RCHUNK)], out_hbm.at[c].at[rr])


def _make_sc_layer():
    mesh = plsc.VectorSubcoreMesh(core_axis_name="c", subcore_axis_name="s")
    return pl.kernel(
        _sc_layer_body,
        mesh=mesh,
        out_type=jax.ShapeDtypeStruct((NCORE, NP, D), jnp.float32),
        scratch_types=[
            pltpu.VMEM((2, A, CHUNK), jnp.int32),
            pltpu.VMEM((2, A * CHUNK), jnp.int32),
            pltpu.VMEM((2, CHUNK), jnp.int32),
            pltpu.VMEM((2, CHUNK), jnp.int32),
            pltpu.VMEM((2, (A // 2) * CHUNK), jnp.int32),
            pltpu.VMEM((2, A * CHUNK, D), jnp.float32),
            pltpu.VMEM((CHUNK, D), jnp.float32),
            pltpu.VMEM((CHUNK, D), jnp.float32),
            pltpu.VMEM((A * CHUNK, D), jnp.float32),
            pltpu.VMEM((A, D), jnp.float32),
            pltpu.VMEM_SHARED((NP, D), jnp.float32),
            pltpu.SemaphoreType.DMA,
            pltpu.SemaphoreType.DMA,
            pltpu.SemaphoreType.DMA,
            pltpu.SemaphoreType.DMA,
            pltpu.SemaphoreType.DMA,
            pltpu.SemaphoreType.DMA,
        ],
    )


def _init_body(tgt_ref, vals_ref, o_ref):
    o_ref[...] = jnp.zeros(o_ref.shape, o_ref.dtype)
    B = tgt_ref.shape[0]
    for b in range(B):
        for j in range(A):
            t = tgt_ref[b, j]
            cur = o_ref[b, pl.ds(t, 1), :]
            o_ref[b, pl.ds(t, 1), :] = cur + vals_ref[b, j, :][None, :]


def _dense_body(agg_ref, xp_ref, w_ref, b_ref, sc_ref, bi_ref, o_ref):
    h = jnp.dot(agg_ref[...], w_ref[...],
                preferred_element_type=jnp.float32) + b_ref[...]
    mu = jnp.mean(h, axis=-1, keepdims=True)
    hc = h - mu
    var = jnp.mean(hc * hc, axis=-1, keepdims=True)
    h = hc * lax.rsqrt(var + 1e-5) * sc_ref[...] + bi_ref[...]
    blk = agg_ref.shape[0]
    rowid = (lax.broadcasted_iota(jnp.int32, (blk, 1), 0)
             + pl.program_id(0) * blk)
    keep = ((rowid % NP) != 0).astype(jnp.float32)
    o_ref[...] = (jnp.maximum(h, 0.0) + xp_ref[...]) * keep


def _head_body(coll_ref, x_ref, q_ref, w1_ref, b1_ref, w2_ref, o_ref, xg_ref):
    B, NC = coll_ref.shape
    for b in range(B):
        for n in range(NC):
            t = coll_ref[b, n]
            xg_ref[pl.ds(b * NC + n, 1), :] = x_ref[b, pl.ds(t, 1), :]
    h1 = jnp.dot(xg_ref[...], w1_ref[:D, :], preferred_element_type=jnp.float32)
    qc = jnp.dot(q_ref[...], w1_ref[D:, :], preferred_element_type=jnp.float32)
    qfull = jnp.concatenate(
        [jnp.broadcast_to(qc[b:b + 1, :], (NC, qc.shape[1])) for b in range(B)],
        axis=0)
    hmid = jnp.maximum(h1 + qfull + b1_ref[...], 0.0)
    score = jnp.dot(hmid, w2_ref[...], preferred_element_type=jnp.float32)
    o_ref[...] = jnp.broadcast_to(score, o_ref.shape)


@jax.jit
def kernel(r_idx, entities_idx, arity, edge_list, rel_list, query_emb, pos_emb,
           rel_embs, Ws, bs, ln_scales, ln_biases, mlpW1, mlpb1, mlpW2, mlpb2):
    B = r_idx.shape[0]
    E = edge_list.shape[0]
    NL = rel_embs.shape[0]

    all_idx = jnp.transpose(entities_idx, (0, 2, 1))
    mask_for_diff = jnp.all(all_idx[:, :, :1] == all_idx, axis=-1)
    pos_idx = jnp.argmax((~mask_for_diff).astype(jnp.int32), axis=1)
    query = query_emb[r_idx[:, 0]]
    rng = jnp.arange(A)[None, :]
    result = ((rng < arity[:, :1]) & (rng != pos_idx[:, None])).astype(jnp.int32)
    tgt = all_idx[:, :, 0] * result
    vals = query[:, None, :] + pos_emb[result * jnp.arange(1, A + 1)[None, :]]
    vals = vals * (tgt != 0)[:, :, None].astype(jnp.float32)
    collapsed = jnp.take_along_axis(
        all_idx,
        jnp.broadcast_to(pos_idx[:, None, None], (B, 1, all_idx.shape[2])),
        axis=1)[:, 0, :]

    edgeT = jnp.zeros((A, EP), jnp.int32).at[:, :E].set(edge_list.T)
    rel_pad = jnp.zeros((EP,), jnp.int32).at[:E].set(rel_list)
    pos_rows = pos_emb[1:A + 1]

    x = pl.pallas_call(
        _init_body,
        in_specs=[pl.BlockSpec(memory_space=pltpu.SMEM), pl.BlockSpec()],
        out_shape=jax.ShapeDtypeStruct((B, NP, D), jnp.float32),
    )(tgt, vals)

    sc_layer = _make_sc_layer()

    BLK = 256
    nrows = B * NP
    dense = pl.pallas_call(
        _dense_body,
        grid=(nrows // BLK,),
        in_specs=[
            pl.BlockSpec((BLK, D), lambda i: (i, 0)),
            pl.BlockSpec((BLK, D), lambda i: (i, 0)),
            pl.BlockSpec((D, D), lambda i: (0, 0)),
            pl.BlockSpec((1, D), lambda i: (0, 0)),
            pl.BlockSpec((1, D), lambda i: (0, 0)),
            pl.BlockSpec((1, D), lambda i: (0, 0)),
        ],
        out_specs=pl.BlockSpec((BLK, D), lambda i: (i, 0)),
        out_shape=jax.ShapeDtypeStruct((nrows, D), jnp.float32),
    )

    bits = ((jnp.arange(64)[:, None] >> jnp.arange(A)[None, :]) & 1)
    possum = bits.astype(jnp.float32) @ pos_rows
    for l in range(NL):
        agg = sc_layer(x.reshape(B * NP, D), edgeT, rel_pad, rel_embs[l],
                       pos_rows, possum)
        xf = dense(agg.reshape(nrows, D), x.reshape(nrows, D), Ws[l],
                   bs[l][None, :], ln_scales[l][None, :], ln_biases[l][None, :])
        x = xf.reshape(B, NP, D)

    NC = collapsed.shape[1]
    FEAT = mlpW1.shape[0]
    head = pl.pallas_call(
        _head_body,
        in_specs=[pl.BlockSpec(memory_space=pltpu.SMEM)] +
                 [pl.BlockSpec()] * 5,
        out_shape=jax.ShapeDtypeStruct((B * NC, D), jnp.float32),
        scratch_shapes=[pltpu.VMEM((B * NC, D), jnp.float32)],
    )
    sc_out = head(collapsed, x, query, mlpW1, mlpb1[None, :], mlpW2)
    return sc_out[:, 0].reshape(B, NC) + mlpb2[0]
